# baseline (speedup 1.0000x reference)
import functools

import jax
import jax.numpy as jnp
from jax.experimental import pallas as pl
from jax.experimental.pallas import tpu as pltpu


def _shift_lanes(p, off):
    bb, c = p.shape
    if off == 0:
        return p
    z = jnp.zeros((bb, abs(off)), jnp.float32)
    if off > 0:
        return jnp.concatenate([z, p[:, : c - off]], axis=1)
    return jnp.concatenate([p[:, -off:], z], axis=1)


def _eca_kernel_4d(x_ref, w_ref, o_ref, *, k):
    xb = x_ref[...]
    pooled_h = jnp.sum(xb.astype(jnp.float32), axis=-2)
    pooled = jnp.sum(pooled_h, axis=-1)
    pad = (k - 1) // 2
    z = jnp.zeros_like(pooled)
    for j in range(k):
        z = z + w_ref[j : j + 1, :] * _shift_lanes(pooled, pad - j)
    gate = jax.nn.sigmoid(z)
    o_ref[...] = xb * gate.astype(xb.dtype)[:, :, None, None]


def kernel(x_nchw, weight):
    b, c, h, w = x_nchw.shape
    k = weight.shape[-1]
    w_kc = weight.reshape(c, k).T.astype(jnp.float32) / jnp.float32(h * w)

    return pl.pallas_call(
        functools.partial(_eca_kernel_4d, k=k),
        out_shape=jax.ShapeDtypeStruct((b, c, h, w), x_nchw.dtype),
        grid=(b,),
        in_specs=[
            pl.BlockSpec((1, c, h, w), lambda i: (i, 0, 0, 0)),
            pl.BlockSpec((k, c), lambda i: (0, 0)),
        ],
        out_specs=pl.BlockSpec((1, c, h, w), lambda i: (i, 0, 0, 0)),
        compiler_params=pltpu.CompilerParams(
            dimension_semantics=("parallel",),
            vmem_limit_bytes=56 * 1024 * 1024,
        ),
    )(x_nchw, w_kc)
